# Initial kernel scaffold; baseline (speedup 1.0000x reference)
import jax
import jax.numpy as jnp
from jax.experimental import pallas as pl


def kernel(x, in_degree, out_degree, atom_table, in_deg_table, out_deg_table, graph_token):
    raise NotImplementedError("write your pallas kernel here")


# speedup vs baseline: 2.4352x; 2.4352x over previous
import functools

import jax
import jax.numpy as jnp
from jax import lax
from jax.experimental import pallas as pl
from jax.experimental.pallas import tpu as pltpu
from jax.experimental.pallas import tpu_sc as plsc

B, N, F, D = 64, 256, 9, 768
NC, NS, L = 2, 16, 16
NW = NC * NS
R = B * N
RPW = R // NW
C = 4
NCHUNK = RPW // C
OUT_ROWS = B * (N + 1)

_mesh = plsc.VectorSubcoreMesh(core_axis_name="c", subcore_axis_name="s")


@functools.partial(
    pl.kernel,
    out_type=jax.ShapeDtypeStruct((OUT_ROWS, D), jnp.float32),
    mesh=_mesh,
    compiler_params=pltpu.CompilerParams(use_tc_tiling_on_sc=False),
    scratch_types=[
        pltpu.VMEM((NCHUNK, C * F), jnp.int32),
        pltpu.VMEM((NCHUNK, C), jnp.int32),
        pltpu.VMEM((NCHUNK, C), jnp.int32),
        pltpu.VMEM((C * F, D), jnp.float32),
        pltpu.VMEM((C, D), jnp.float32),
        pltpu.VMEM((C, D), jnp.float32),
        pltpu.VMEM((C, D), jnp.float32),
        pltpu.VMEM((1, D), jnp.float32),
        pltpu.SemaphoreType.DMA,
    ],
)
def _sc_body(x_hbm, ind_hbm, outd_hbm, atab, itab, otab, tok, out_hbm,
             x_v, ind_v, outd_v, arows, irows, orows, out_v, tok_v, sem):
    w = lax.axis_index("s") * NC + lax.axis_index("c")

    pltpu.sync_copy(x_hbm.at[w], x_v)
    pltpu.sync_copy(ind_hbm.at[w], ind_v)
    pltpu.sync_copy(outd_hbm.at[w], outd_v)
    pltpu.sync_copy(tok, tok_v)
    b0 = w * (RPW // N)
    for k in range(RPW // N):
        pltpu.sync_copy(tok_v, out_hbm.at[pl.ds((b0 + k) * (N + 1), 1)])

    @pl.loop(0, NCHUNK)
    def _chunk(c):
        pltpu.async_copy(atab.at[x_v.at[c]], arows, sem).wait()
        pltpu.async_copy(itab.at[ind_v.at[c]], irows, sem).wait()
        pltpu.async_copy(otab.at[outd_v.at[c]], orows, sem).wait()

        @pl.loop(0, D // L)
        def _cols(j):
            sl = pl.ds(j * L, L)
            for i in range(C):
                acc = irows[i, sl] + orows[i, sl]
                for f in range(F):
                    acc = acc + arows[i * F + f, sl]
                out_v[i, sl] = acc

        r0 = w * RPW + c * C
        orow = r0 + r0 // N + 1
        pltpu.sync_copy(out_v, out_hbm.at[pl.ds(orow, C)])


def kernel(x, in_degree, out_degree, atom_table, in_deg_table, out_deg_table,
           graph_token):
    x3 = x.reshape(NW, NCHUNK, C * F)
    ind3 = in_degree.reshape(NW, NCHUNK, C)
    outd3 = out_degree.reshape(NW, NCHUNK, C)
    out = _sc_body(x3, ind3, outd3, atom_table, in_deg_table, out_deg_table,
                   graph_token)
    return out.reshape(B, N + 1, D)


# speedup vs baseline: 4.8347x; 1.9854x over previous
import functools

import jax
import jax.numpy as jnp
from jax import lax
from jax.experimental import pallas as pl
from jax.experimental.pallas import tpu as pltpu
from jax.experimental.pallas import tpu_sc as plsc

B, N, F, D = 64, 256, 9, 768
NC, NS, L = 2, 16, 16
NW = NC * NS
R = B * N
RPW = R // NW
C = 4
NCHUNK = RPW // C
OUT_ROWS = B * (N + 1)

_mesh = plsc.VectorSubcoreMesh(core_axis_name="c", subcore_axis_name="s")


@functools.partial(
    pl.kernel,
    out_type=jax.ShapeDtypeStruct((OUT_ROWS, D), jnp.float32),
    mesh=_mesh,
    compiler_params=pltpu.CompilerParams(use_tc_tiling_on_sc=False),
    scratch_types=[
        pltpu.VMEM((NCHUNK, C * F), jnp.int32),
        pltpu.VMEM((NCHUNK, C), jnp.int32),
        pltpu.VMEM((NCHUNK, C), jnp.int32),
        pltpu.VMEM((2, C * F, D), jnp.float32),
        pltpu.VMEM((2, C, D), jnp.float32),
        pltpu.VMEM((2, C, D), jnp.float32),
        pltpu.VMEM((2, C, D), jnp.float32),
        pltpu.VMEM((1, D), jnp.float32),
        pltpu.SemaphoreType.DMA,
        pltpu.SemaphoreType.DMA,
        pltpu.SemaphoreType.DMA,
        pltpu.SemaphoreType.DMA,
    ],
)
def _sc_body(x_hbm, ind_hbm, outd_hbm, atab, itab, otab, tok, out_hbm,
             x_v, ind_v, outd_v, arows, irows, orows, out_v, tok_v,
             semg0, semg1, semo0, semo1):
    w = lax.axis_index("s") * NC + lax.axis_index("c")
    semg = (semg0, semg1)
    semo = (semo0, semo1)

    pltpu.sync_copy(x_hbm.at[w], x_v)
    pltpu.sync_copy(ind_hbm.at[w], ind_v)
    pltpu.sync_copy(outd_hbm.at[w], outd_v)
    pltpu.sync_copy(tok, tok_v)
    b0 = w * (RPW // N)
    for k in range(RPW // N):
        pltpu.sync_copy(tok_v, out_hbm.at[pl.ds((b0 + k) * (N + 1), 1)])

    def fire_gathers(c, p):
        pltpu.async_copy(atab.at[x_v.at[c]], arows.at[p], semg[p])
        pltpu.async_copy(itab.at[ind_v.at[c]], irows.at[p], semg[p])
        pltpu.async_copy(otab.at[outd_v.at[c]], orows.at[p], semg[p])

    def wait_gathers(c, p):
        pltpu.make_async_copy(atab.at[x_v.at[c]], arows.at[p], semg[p]).wait()
        pltpu.make_async_copy(itab.at[ind_v.at[c]], irows.at[p], semg[p]).wait()
        pltpu.make_async_copy(otab.at[outd_v.at[c]], orows.at[p], semg[p]).wait()

    def out_row(c):
        r0 = w * RPW + c * C
        return r0 + r0 // N + 1

    def out_copy(c, p):
        return pltpu.make_async_copy(
            out_v.at[p], out_hbm.at[pl.ds(out_row(c), C)], semo[p])

    fire_gathers(0, 0)

    @pl.loop(0, NCHUNK, step=2)
    def _c2(c0):
        for p in range(2):
            c = c0 + p
            q = 1 - p

            @pl.when(c + 1 < NCHUNK)
            def _():
                fire_gathers(c + 1, q)

            wait_gathers(c, p)

            @pl.when(c >= 2)
            def _():
                out_copy(c - 2, p).wait()

            @pl.loop(0, D // L)
            def _cols(j):
                sl = pl.ds(j * L, L)
                for i in range(C):
                    acc = irows[p, i, sl] + orows[p, i, sl]
                    for f in range(F):
                        acc = acc + arows[p, i * F + f, sl]
                    out_v[p, i, sl] = acc

            out_copy(c, p).start()

    out_copy(NCHUNK - 2, 0).wait()
    out_copy(NCHUNK - 1, 1).wait()


def kernel(x, in_degree, out_degree, atom_table, in_deg_table, out_deg_table,
           graph_token):
    x3 = x.reshape(NW, NCHUNK, C * F)
    ind3 = in_degree.reshape(NW, NCHUNK, C)
    outd3 = out_degree.reshape(NW, NCHUNK, C)
    out = _sc_body(x3, ind3, outd3, atom_table, in_deg_table, out_deg_table,
                   graph_token)
    return out.reshape(B, N + 1, D)
